# initial kernel scaffold (unmeasured)
import jax
import jax.numpy as jnp
from jax import lax
from jax.experimental import pallas as pl
from jax.experimental.pallas import tpu as pltpu

N_DEV = 32


def kernel(x, w_mat, scale_x, scale_w):
    m_per, k = x.shape
    _, n_per = w_mat.shape
    m_glob = N_DEV * m_per

    def body(x_ref, w_ref, sx_ref, sw_ref, out_ref,
             gather_ref, send_sems, recv_sems):
        my = lax.axis_index("i")
        left = lax.rem(my - 1 + N_DEV, N_DEV)
        right = lax.rem(my + 1, N_DEV)

        barrier_sem = pltpu.get_barrier_semaphore()
        for nbr in (left, right):
            pl.semaphore_signal(
                barrier_sem, inc=1,
                device_id=(nbr,), device_id_type=pl.DeviceIdType.MESH,
            )
        pl.semaphore_wait(barrier_sem, 2)

        scale = sx_ref[0] * sw_ref[0]

        def compute_chunk(origin):
            rows = pl.ds(origin * m_per, m_per)
            chunk = gather_ref[rows, :]
            acc = jax.lax.dot_general(
                chunk, w_ref[...],
                dimension_numbers=(((1,), (0,)), ((), ())),
                preferred_element_type=jnp.float32,
            )
            y = acc * scale
            out_ref[rows, :] = y * (1.0 / (1.0 + jnp.exp(-y)))

        my_rows = pl.ds(my * m_per, m_per)
        gather_ref[my_rows, :] = x_ref[...]
        compute_chunk(my)

        for h in range(N_DEV - 1):
            send_origin = lax.rem(my - h + N_DEV, N_DEV)
            send_rows = pl.ds(send_origin * m_per, m_per)
            rdma = pltpu.make_async_remote_copy(
                src_ref=gather_ref.at[send_rows, :],
                dst_ref=gather_ref.at[send_rows, :],
                send_sem=send_sems.at[h],
                recv_sem=recv_sems.at[h],
                device_id=(right,),
                device_id_type=pl.DeviceIdType.MESH,
            )
            rdma.start()
            rdma.wait()

            recv_origin = lax.rem(my - h - 1 + N_DEV, N_DEV)
            compute_chunk(recv_origin)

    return pl.pallas_call(
        body,
        out_shape=jax.ShapeDtypeStruct((m_glob, n_per), jnp.float32),
        in_specs=[
            pl.BlockSpec(memory_space=pltpu.VMEM),
            pl.BlockSpec(memory_space=pltpu.VMEM),
            pl.BlockSpec(memory_space=pltpu.SMEM),
            pl.BlockSpec(memory_space=pltpu.SMEM),
        ],
        out_specs=pl.BlockSpec(memory_space=pltpu.VMEM),
        scratch_shapes=[
            pltpu.VMEM((m_glob, k), x.dtype),
            pltpu.SemaphoreType.DMA((N_DEV - 1,)),
            pltpu.SemaphoreType.DMA((N_DEV - 1,)),
        ],
        compiler_params=pltpu.CompilerParams(collective_id=0),
    )(x, w_mat, scale_x, scale_w)


# baseline (device time: 249020 ns/iter reference)
import jax
import jax.numpy as jnp
from jax import lax
from jax.experimental import pallas as pl
from jax.experimental.pallas import tpu as pltpu

N_DEV = 32


def kernel(x, w_mat, scale_x, scale_w):
    m_per, k = x.shape
    _, n_per = w_mat.shape
    m_glob = N_DEV * m_per

    x = x.astype(jnp.float8_e4m3fn)
    w_mat = w_mat.astype(jnp.float8_e4m3fn)

    def body(x_ref, w_ref, sx_ref, sw_ref, out_ref,
             gather_ref, send_sems, recv_sems):
        my = lax.axis_index("i")
        left = lax.rem(my - 1 + N_DEV, N_DEV)
        right = lax.rem(my + 1, N_DEV)

        barrier_sem = pltpu.get_barrier_semaphore()
        for nbr in (left, right):
            pl.semaphore_signal(
                barrier_sem, inc=1,
                device_id=(nbr,), device_id_type=pl.DeviceIdType.MESH,
            )
        pl.semaphore_wait(barrier_sem, 2)

        scale = sx_ref[0] * sw_ref[0]

        def compute_chunk(origin):
            rows = pl.ds(origin * m_per, m_per)
            chunk = gather_ref[rows, :]
            acc = jax.lax.dot_general(
                chunk, w_ref[...],
                dimension_numbers=(((1,), (0,)), ((), ())),
                preferred_element_type=jnp.float32,
            )
            y = acc * scale
            out_ref[rows, :] = y * (1.0 / (1.0 + jnp.exp(-y)))

        my_rows = pl.ds(my * m_per, m_per)
        gather_ref[my_rows, :] = x_ref[...]
        compute_chunk(my)

        for h in range(N_DEV - 1):
            send_origin = lax.rem(my - h + N_DEV, N_DEV)
            send_rows = pl.ds(send_origin * m_per, m_per)
            rdma = pltpu.make_async_remote_copy(
                src_ref=gather_ref.at[send_rows, :],
                dst_ref=gather_ref.at[send_rows, :],
                send_sem=send_sems.at[h],
                recv_sem=recv_sems.at[h],
                device_id=(right,),
                device_id_type=pl.DeviceIdType.MESH,
            )
            rdma.start()
            rdma.wait()

            recv_origin = lax.rem(my - h - 1 + N_DEV, N_DEV)
            compute_chunk(recv_origin)

    return pl.pallas_call(
        body,
        out_shape=jax.ShapeDtypeStruct((m_glob, n_per), jnp.float32),
        in_specs=[
            pl.BlockSpec(memory_space=pltpu.VMEM),
            pl.BlockSpec(memory_space=pltpu.VMEM),
            pl.BlockSpec(memory_space=pltpu.SMEM),
            pl.BlockSpec(memory_space=pltpu.SMEM),
        ],
        out_specs=pl.BlockSpec(memory_space=pltpu.VMEM),
        scratch_shapes=[
            pltpu.VMEM((m_glob, k), x.dtype),
            pltpu.SemaphoreType.DMA((N_DEV - 1,)),
            pltpu.SemaphoreType.DMA((N_DEV - 1,)),
        ],
        compiler_params=pltpu.CompilerParams(collective_id=0),
    )(x, w_mat, scale_x, scale_w)


# device time: 126514 ns/iter; 1.9683x vs baseline; 1.9683x over previous
import jax
import jax.numpy as jnp
import numpy as np
from jax import lax
from jax.experimental import pallas as pl
from jax.experimental.pallas import tpu as pltpu

N_DEV = 32
HF = 16
HB = 15


def _logical_coords():
    order = []
    for z in range(4):
        for yi in range(4):
            row = [(x, yi, z) for x in range(2)]
            if yi % 2:
                row = row[::-1]
            order.extend(row)
    return order


def _hamiltonian_cycle():
    path0 = []
    for zi in range(4):
        ys = range(4) if zi % 2 == 0 else range(3, -1, -1)
        for y in ys:
            path0.append((0, y, z := zi))
    path1 = [(1, y, z) for (_, y, z) in reversed(path0)]
    return path0 + path1


_COORD_TO_LOGICAL = {c: i for i, c in enumerate(_logical_coords())}
CYCLE = np.array([_COORD_TO_LOGICAL[c] for c in _hamiltonian_cycle()],
                 dtype=np.int32)
POS = np.empty(N_DEV, dtype=np.int32)
POS[CYCLE] = np.arange(N_DEV, dtype=np.int32)


def kernel(x, w_mat, scale_x, scale_w):
    m_per, k = x.shape
    _, n_per = w_mat.shape
    m_glob = N_DEV * m_per

    x = x.astype(jnp.float8_e4m3fn)
    w_mat = w_mat.astype(jnp.float8_e4m3fn)

    def body(x_ref, w_ref, sx_ref, sw_ref, cyc_ref, pos_ref, out_ref,
             gather_ref, fsend, frecv, bsend, brecv, dummy):
        my = lax.axis_index("i")
        r = pos_ref[my]
        right = cyc_ref[lax.rem(r + 1, N_DEV)]
        left = cyc_ref[lax.rem(r - 1 + N_DEV, N_DEV)]

        barrier_sem = pltpu.get_barrier_semaphore()
        for nbr in (left, right):
            pl.semaphore_signal(
                barrier_sem, inc=1,
                device_id=(nbr,), device_id_type=pl.DeviceIdType.MESH,
            )
        pl.semaphore_wait(barrier_sem, 2)

        scale = sx_ref[0] * sw_ref[0]

        def compute(chunk_ref, origin):
            acc = jax.lax.dot_general(
                chunk_ref[...], w_ref[...],
                dimension_numbers=(((1,), (0,)), ((), ())),
                preferred_element_type=jnp.float32,
            )
            y = acc * scale
            out_ref[pl.ds(origin * m_per, m_per), :] = (
                y * (1.0 / (1.0 + jnp.exp(-y)))
            )

        def rows(origin):
            return pl.ds(origin * m_per, m_per)

        def make(src, dst_origin, send_sem, recv_sem, dev):
            return pltpu.make_async_remote_copy(
                src_ref=src,
                dst_ref=gather_ref.at[rows(dst_origin), :],
                send_sem=send_sem,
                recv_sem=recv_sem,
                device_id=(dev,),
                device_id_type=pl.DeviceIdType.MESH,
            )

        gather_ref[rows(my), :] = x_ref[...]
        f_prev = make(gather_ref.at[rows(my), :], my,
                      fsend.at[0], frecv.at[0], right)
        f_prev.start()
        b_prev = make(gather_ref.at[rows(my), :], my,
                      bsend.at[0], brecv.at[0], left)
        b_prev.start()
        compute(x_ref, my)

        for h in range(HF):
            o_f = cyc_ref[lax.rem(r - 1 - h + 2 * N_DEV, N_DEV)]
            src_f = gather_ref.at[rows(o_f), :]
            rcv = make(src_f, o_f, dummy.at[0], frecv.at[h], left)
            rcv.wait_recv()
            if h + 1 < HF:
                snd = make(src_f, o_f, fsend.at[h + 1], frecv.at[h + 1],
                           right)
                snd.start()
                f_prev.wait_send()
                f_prev = snd
            compute(src_f, o_f)

            if h < HB:
                o_b = cyc_ref[lax.rem(r + 1 + h, N_DEV)]
                src_b = gather_ref.at[rows(o_b), :]
                rcvb = make(src_b, o_b, dummy.at[0], brecv.at[h], right)
                rcvb.wait_recv()
                if h + 1 < HB:
                    sndb = make(src_b, o_b, bsend.at[h + 1],
                                brecv.at[h + 1], left)
                    sndb.start()
                    b_prev.wait_send()
                    b_prev = sndb
                compute(src_b, o_b)

        f_prev.wait_send()
        b_prev.wait_send()

    return pl.pallas_call(
        body,
        out_shape=jax.ShapeDtypeStruct((m_glob, n_per), jnp.float32),
        in_specs=[
            pl.BlockSpec(memory_space=pltpu.VMEM),
            pl.BlockSpec(memory_space=pltpu.VMEM),
            pl.BlockSpec(memory_space=pltpu.SMEM),
            pl.BlockSpec(memory_space=pltpu.SMEM),
            pl.BlockSpec(memory_space=pltpu.SMEM),
            pl.BlockSpec(memory_space=pltpu.SMEM),
        ],
        out_specs=pl.BlockSpec(memory_space=pltpu.VMEM),
        scratch_shapes=[
            pltpu.VMEM((m_glob, k), x.dtype),
            pltpu.SemaphoreType.DMA((HF,)),
            pltpu.SemaphoreType.DMA((HF,)),
            pltpu.SemaphoreType.DMA((HB,)),
            pltpu.SemaphoreType.DMA((HB,)),
            pltpu.SemaphoreType.DMA((1,)),
        ],
        compiler_params=pltpu.CompilerParams(collective_id=0),
    )(x, w_mat, scale_x, scale_w, jnp.asarray(CYCLE), jnp.asarray(POS))


# device time: 100262 ns/iter; 2.4837x vs baseline; 1.2618x over previous
import jax
import jax.numpy as jnp
import numpy as np
from jax import lax
from jax.experimental import pallas as pl
from jax.experimental.pallas import tpu as pltpu

N_DEV = 32
HF = 16
HB = 15


def _logical_coords():
    order = []
    for z in range(4):
        for yi in range(4):
            row = [(x, yi, z) for x in range(2)]
            if yi % 2:
                row = row[::-1]
            order.extend(row)
    return order


def _hamiltonian_cycle():
    path0 = []
    for zi in range(4):
        ys = range(4) if zi % 2 == 0 else range(3, -1, -1)
        for y in ys:
            path0.append((0, y, z := zi))
    path1 = [(1, y, z) for (_, y, z) in reversed(path0)]
    return path0 + path1


_COORD_TO_LOGICAL = {c: i for i, c in enumerate(_logical_coords())}
CYCLE = np.array([_COORD_TO_LOGICAL[c] for c in _hamiltonian_cycle()],
                 dtype=np.int32)
POS = np.empty(N_DEV, dtype=np.int32)
POS[CYCLE] = np.arange(N_DEV, dtype=np.int32)


def kernel(x, w_mat, scale_x, scale_w):
    m_per, k = x.shape
    _, n_per = w_mat.shape
    m_glob = N_DEV * m_per

    x = x.astype(jnp.float8_e4m3fn)
    w_mat = w_mat.astype(jnp.float8_e4m3fn)

    def body(x_ref, w_ref, sx_ref, sw_ref, cyc_ref, pos_ref, out_ref,
             gather_ref, fsend, frecv, bsend, brecv, dummy):
        my = lax.axis_index("i")
        r = pos_ref[my]
        right = cyc_ref[lax.rem(r + 1, N_DEV)]
        left = cyc_ref[lax.rem(r - 1 + N_DEV, N_DEV)]

        barrier_sem = pltpu.get_barrier_semaphore()
        for nbr in (left, right):
            pl.semaphore_signal(
                barrier_sem, inc=1,
                device_id=(nbr,), device_id_type=pl.DeviceIdType.MESH,
            )
        pl.semaphore_wait(barrier_sem, 2)

        scale = sx_ref[0] * sw_ref[0]

        def compute(chunk_ref, origin):
            acc = jax.lax.dot_general(
                chunk_ref[...], w_ref[...],
                dimension_numbers=(((1,), (0,)), ((), ())),
                preferred_element_type=jnp.float32,
            )
            y = acc * scale
            out_ref[pl.ds(origin * m_per, m_per), :] = (
                y * (1.0 / (1.0 + jnp.exp(-y)))
            )

        m_sub = m_per // 2

        def rows(origin):
            return pl.ds(origin * m_per, m_per)

        def sub_rows(origin, j):
            return pl.ds(origin * m_per + j * m_sub, m_sub)

        def make(origin, j, send_sem, recv_sem, dev):
            sl = gather_ref.at[sub_rows(origin, j), :]
            return pltpu.make_async_remote_copy(
                src_ref=sl,
                dst_ref=sl,
                send_sem=send_sem,
                recv_sem=recv_sem,
                device_id=(dev,),
                device_id_type=pl.DeviceIdType.MESH,
            )

        gather_ref[rows(my), :] = x_ref[...]
        f_prev = [None, None]
        b_prev = [None, None]
        for j in range(2):
            f_prev[j] = make(my, j, fsend.at[0, j], frecv.at[0, j], right)
            f_prev[j].start()
            b_prev[j] = make(my, j, bsend.at[0, j], brecv.at[0, j], left)
            b_prev[j].start()
        compute(x_ref, my)

        for h in range(HF):
            o_f = cyc_ref[lax.rem(r - 1 - h + 2 * N_DEV, N_DEV)]
            for j in range(2):
                rcv = make(o_f, j, dummy.at[0], frecv.at[h, j], left)
                rcv.wait_recv()
                if h + 1 < HF:
                    snd = make(o_f, j, fsend.at[h + 1, j],
                               frecv.at[h + 1, j], right)
                    snd.start()
                    f_prev[j].wait_send()
                    f_prev[j] = snd
            compute(gather_ref.at[rows(o_f), :], o_f)

            if h < HB:
                o_b = cyc_ref[lax.rem(r + 1 + h, N_DEV)]
                for j in range(2):
                    rcvb = make(o_b, j, dummy.at[0], brecv.at[h, j], right)
                    rcvb.wait_recv()
                    if h + 1 < HB:
                        sndb = make(o_b, j, bsend.at[h + 1, j],
                                    brecv.at[h + 1, j], left)
                        sndb.start()
                        b_prev[j].wait_send()
                        b_prev[j] = sndb
                compute(gather_ref.at[rows(o_b), :], o_b)

        for j in range(2):
            f_prev[j].wait_send()
            b_prev[j].wait_send()

    return pl.pallas_call(
        body,
        out_shape=jax.ShapeDtypeStruct((m_glob, n_per), jnp.float32),
        in_specs=[
            pl.BlockSpec(memory_space=pltpu.VMEM),
            pl.BlockSpec(memory_space=pltpu.VMEM),
            pl.BlockSpec(memory_space=pltpu.SMEM),
            pl.BlockSpec(memory_space=pltpu.SMEM),
            pl.BlockSpec(memory_space=pltpu.SMEM),
            pl.BlockSpec(memory_space=pltpu.SMEM),
        ],
        out_specs=pl.BlockSpec(memory_space=pltpu.VMEM),
        scratch_shapes=[
            pltpu.VMEM((m_glob, k), x.dtype),
            pltpu.SemaphoreType.DMA((HF, 2)),
            pltpu.SemaphoreType.DMA((HF, 2)),
            pltpu.SemaphoreType.DMA((HB, 2)),
            pltpu.SemaphoreType.DMA((HB, 2)),
            pltpu.SemaphoreType.DMA((1,)),
        ],
        compiler_params=pltpu.CompilerParams(collective_id=0),
    )(x, w_mat, scale_x, scale_w, jnp.asarray(CYCLE), jnp.asarray(POS))
